# Initial kernel scaffold; baseline (speedup 1.0000x reference)
import jax
import jax.numpy as jnp
from jax.experimental import pallas as pl


def kernel(seq, r, j_idx, h, r_half_raw, tau_hp_raw):
    raise NotImplementedError("write your pallas kernel here")


# speedup vs baseline: 429.5884x; 429.5884x over previous
import functools

import jax
import jax.numpy as jnp
import numpy as np
from jax import lax
from jax.experimental import pallas as pl
from jax.experimental.pallas import tpu as pltpu
from jax.experimental.pallas import tpu_sc as plsc

_LANES = 16
_MAX_DIST = 10.0
_VALID_THRESH = float(np.float32(10.0 - 0.0001))


def _make_sc_kernel(B, L, K, n_workers):
    rows_per_w = (B * L) // n_workers
    w_per_b = n_workers // B
    CH = 256
    n_chunks = rows_per_w // CH
    n_blocks = CH // _LANES
    ht_chunks = L // _LANES

    mesh = plsc.VectorSubcoreMesh(core_axis_name="c", subcore_axis_name="s",
                                  num_cores=2, num_subcores=16)

    @functools.partial(
        pl.kernel,
        out_type=jax.ShapeDtypeStruct((B, L), jnp.float32),
        mesh=mesh,
        compiler_params=pltpu.CompilerParams(needs_layout_passes=False),
        scratch_types=[
            pltpu.VMEM((128,), jnp.float32),
            pltpu.VMEM((32,), jnp.float32),
            pltpu.VMEM((L,), jnp.int32),
            pltpu.VMEM((L,), jnp.float32),
            pltpu.VMEM((CH, K), jnp.int32),
            pltpu.VMEM((CH, K), jnp.float32),
            pltpu.VMEM((CH,), jnp.float32),
        ],
    )
    def sc_kernel(seq_hbm, r_hbm, j_hbm, h_hbm, par_hbm, out_hbm,
                  h_v, par_v, seq_v, hf_v, j_v, r_v, o_v):
        nc = mesh.num_cores
        wid = lax.axis_index("s") * nc + lax.axis_index("c")
        b = wid // w_per_b
        half = wid % w_per_b

        pltpu.sync_copy(h_hbm, h_v)
        pltpu.sync_copy(par_hbm, par_v)
        pltpu.sync_copy(seq_hbm.at[b], seq_v)

        def build(i, carry):
            sv = seq_v[pl.ds(i * _LANES, _LANES)]
            hf_v[pl.ds(i * _LANES, _LANES)] = plsc.load_gather(h_v, [sv])
            return carry
        lax.fori_loop(0, ht_chunks, build, 0)

        iota = lax.iota(jnp.int32, _LANES)
        peak = par_v[pl.ds(0, _LANES)]
        inv2s2 = par_v[pl.ds(_LANES, _LANES)]

        for c in range(n_chunks):
            lo = half * rows_per_w + c * CH
            pltpu.sync_copy(j_hbm.at[b, pl.ds(lo, CH)], j_v)
            pltpu.sync_copy(r_hbm.at[b, pl.ds(lo, CH)], r_v)

            def rowblock(rb, carry):
                rows = iota + rb * _LANES

                def kstep(k, acc):
                    kv = jnp.full((_LANES,), k, jnp.int32)
                    jv = plsc.load_gather(j_v, [rows, kv])
                    rv = plsc.load_gather(r_v, [rows, kv])
                    jv = jnp.clip(jv, 0, L - 1)
                    hv = plsc.load_gather(hf_v, [jv])
                    d = jnp.minimum(rv, _MAX_DIST) - peak
                    g = jnp.exp(-(d * d) * inv2s2)
                    g = jnp.where(rv < _VALID_THRESH, g, 0.0)
                    return acc + hv * g

                acc = lax.fori_loop(0, K, kstep,
                                    jnp.zeros((_LANES,), jnp.float32))
                hi = plsc.load_gather(hf_v, [rows + lo])
                o_v[pl.ds(rb * _LANES, _LANES)] = hi * acc
                return carry
            lax.fori_loop(0, n_blocks, rowblock, 0)
            pltpu.sync_copy(o_v, out_hbm.at[b, pl.ds(lo, CH)])

    return sc_kernel


def kernel(seq, r, j_idx, h, r_half_raw, tau_hp_raw):
    B, L, K = r.shape
    r_peak = jax.nn.softplus(r_half_raw).astype(jnp.float32)
    sigma = (jax.nn.softplus(tau_hp_raw) + 0.1).astype(jnp.float32)
    inv2s2 = 1.0 / (2.0 * sigma * sigma)
    par = jnp.concatenate([
        jnp.full((_LANES,), r_peak, jnp.float32),
        jnp.full((_LANES,), inv2s2, jnp.float32),
    ])
    h_pad = jnp.pad(h.astype(jnp.float32), (0, 128 - h.shape[0]))
    sc = _make_sc_kernel(B, L, K, 32)
    return sc(seq.astype(jnp.int32), r, j_idx.astype(jnp.int32), h_pad, par)
